# baseline (speedup 1.0000x reference)
import jax
import jax.numpy as jnp
from jax import lax
from jax.experimental import pallas as pl
from jax.experimental.pallas import tpu as pltpu
from jax.experimental.pallas import tpu_sc as plsc

N_NODES = 10000
NPAD = 10240
N_EDGES = 320000
D = 128
ALPHA = 0.2
NC, NS, L = 2, 16, 16
NW = NC * NS
EB = 64
UNROLL = 12
NI = 168
E_PAD = NI * NW * EB
TILE_ROWS = NPAD // NS
NRR = 3
NRE = 4


def _scores_body(head_ref, tail_ref, attn_ref, ah_ref, at_ref):
    aw = attn_ref[...]
    ah_ref[...] = jnp.sum(head_ref[...] * aw[:, :D], axis=1, keepdims=True)
    at_ref[...] = jnp.sum(tail_ref[...] * aw[:, D:], axis=1, keepdims=True)


def _sc_body(ah_hbm, at_hbm, tv_hbm, src_hbm, dst_hbm,
             hp_out, rs_out,
             ah_tab, at_tab, sbuf, dbuf, wbuf, rows,
             hp_acc, rs_acc,
             sem_is, sem_id, sem_g, sem_s, sem_r):
    c = lax.axis_index("c")
    s = lax.axis_index("s")
    wid = s * NC + c

    zero16 = jnp.zeros((L,), jnp.float32)

    def _zbody(r, carry):
        for j in range(D // L):
            rows[0, r, pl.ds(j * L, L)] = zero16
        return carry

    lax.fori_loop(0, EB, _zbody, 0)
    for j in range(EB // L):
        wbuf[0, pl.ds(j * L, L)] = zero16

    tbase = s * TILE_ROWS
    for k in range(TILE_ROWS // EB):
        pltpu.sync_copy(rows.at[0], hp_acc.at[pl.ds(tbase + k * EB, EB)])
        pltpu.sync_copy(wbuf.at[0], rs_acc.at[pl.ds(tbase + k * EB, EB)])

    pltpu.sync_copy(ah_hbm, ah_tab)
    pltpu.sync_copy(at_hbm, at_tab)

    plsc.subcore_barrier()

    def _base(i):
        return pl.multiple_of((i * NW + wid) * EB, 32)

    def _issue_idx(i, e):
        pltpu.async_copy(src_hbm.at[pl.ds(_base(i), EB)], sbuf.at[e],
                         sem_is.at[e])
        pltpu.async_copy(dst_hbm.at[pl.ds(_base(i), EB)], dbuf.at[e],
                         sem_id.at[e])

    def _wait_idx(i, e):
        pltpu.make_async_copy(src_hbm.at[pl.ds(_base(i), EB)], sbuf.at[e],
                              sem_is.at[e]).wait()
        pltpu.make_async_copy(dst_hbm.at[pl.ds(_base(i), EB)], dbuf.at[e],
                              sem_id.at[e]).wait()

    def _issue_gather(r, e):
        pltpu.async_copy(tv_hbm.at[dbuf.at[e]], rows.at[r], sem_g.at[r])

    def _wait_gather(r, e):
        pltpu.make_async_copy(tv_hbm.at[dbuf.at[e]], rows.at[r],
                              sem_g.at[r]).wait()

    def _issue_scatters(r, e):
        pltpu.async_copy(rows.at[r], hp_acc.at[sbuf.at[e]], sem_s.at[r],
                         add=True)
        pltpu.async_copy(wbuf.at[r], rs_acc.at[sbuf.at[e]], sem_r.at[r],
                         add=True)

    def _wait_scatters(r, e):
        pltpu.make_async_copy(rows.at[r], hp_acc.at[sbuf.at[e]],
                              sem_s.at[r]).wait()
        pltpu.make_async_copy(wbuf.at[r], rs_acc.at[sbuf.at[e]],
                              sem_r.at[r]).wait()

    def _compute(r, e):
        for j in range(EB // L):
            si = sbuf[e, pl.ds(j * L, L)]
            di = dbuf[e, pl.ds(j * L, L)]
            x = plsc.load_gather(ah_tab, [si]) + plsc.load_gather(at_tab, [di])
            x = jnp.clip(x, -10.0, 10.0)
            x = jnp.where(x >= 0.0, x, ALPHA * x)
            wbuf[r, pl.ds(j * L, L)] = jnp.exp(x)

        def _mbody(g, mcarry):
            wv = wbuf[r, pl.ds(g * L, L)]
            for l in range(L):
                wr = wv[l]
                rr = g * L + l
                for jj in range(D // L):
                    rows[r, rr, pl.ds(jj * L, L)] = (
                        rows[r, rr, pl.ds(jj * L, L)] * wr)
            return mcarry

        lax.fori_loop(0, EB // L, _mbody, 0)

    _issue_idx(0, 0)
    _issue_idx(1, 1)
    _wait_idx(0, 0)
    _issue_gather(0, 0)

    NITER = NI // UNROLL

    def _body(ii, carry):
        for u in range(UNROLL):
            i = ii * UNROLL + u
            r = u % NRR
            e = u % NRE
            r1 = (u + 1) % NRR
            e1 = (u + 1) % NRE
            rm2 = (u - 2) % NRR
            em2 = (u - 2) % NRE
            e2 = (u + 2) % NRE

            if u >= 2:
                _wait_scatters(rm2, em2)
            else:
                @pl.when(ii >= 1)
                def _():
                    _wait_scatters(rm2, em2)

            if u < UNROLL - 2:
                _issue_idx(i + 2, e2)
            else:
                @pl.when(ii < NITER - 1)
                def _():
                    _issue_idx(i + 2, e2)

            def _advance():
                _wait_idx(i + 1, e1)
                _issue_gather(r1, e1)
            if u < UNROLL - 1:
                _advance()
            else:
                @pl.when(ii < NITER - 1)
                def _():
                    _advance()

            _wait_gather(r, e)
            _compute(r, e)
            _issue_scatters(r, e)
        return carry

    lax.fori_loop(0, NITER, _body, 0)

    for ib in (NI - 2, NI - 1):
        _wait_scatters(ib % NRR, ib % NRE)

    plsc.subcore_barrier()

    pltpu.sync_copy(hp_acc.at[pl.ds(tbase, TILE_ROWS)],
                    hp_out.at[c, pl.ds(tbase, TILE_ROWS)])
    pltpu.sync_copy(rs_acc.at[pl.ds(tbase, TILE_ROWS)],
                    rs_out.at[c, pl.ds(tbase, TILE_ROWS)])


_CB = 1024


def _combine_body(hp_ref, rs_ref, hp_out_ref, rs_out_ref):
    hp_out_ref[...] = hp_ref[0] + hp_ref[1]
    rs_out_ref[...] = (rs_ref[0] + rs_ref[1])[:, None]


def kernel(head_rep, tail_rep, tail_val, edge_list, rel_list, attn):
    f32 = jnp.float32
    i32 = jnp.int32
    head_p = jnp.pad(head_rep.astype(f32), ((0, NPAD - N_NODES), (0, 0)))
    tail_p = jnp.pad(tail_rep.astype(f32), ((0, NPAD - N_NODES), (0, 0)))
    pad_n = E_PAD - N_EDGES
    src_p = jnp.concatenate([edge_list[0].astype(i32),
                             jnp.full((pad_n,), N_NODES, dtype=i32)])
    dst_p = jnp.concatenate([edge_list[1].astype(i32),
                             jnp.zeros((pad_n,), dtype=i32)])

    ah2, at2 = pl.pallas_call(
        _scores_body,
        grid=(NPAD // _CB,),
        in_specs=[
            pl.BlockSpec((_CB, D), lambda i: (i, 0)),
            pl.BlockSpec((_CB, D), lambda i: (i, 0)),
            pl.BlockSpec((1, 2 * D), lambda i: (0, 0)),
        ],
        out_specs=[
            pl.BlockSpec((_CB, 1), lambda i: (i, 0)),
            pl.BlockSpec((_CB, 1), lambda i: (i, 0)),
        ],
        out_shape=[
            jax.ShapeDtypeStruct((NPAD, 1), f32),
            jax.ShapeDtypeStruct((NPAD, 1), f32),
        ],
    )(head_p, tail_p, attn.astype(f32))
    ah = ah2.reshape(NPAD)
    at = at2.reshape(NPAD)

    mesh = plsc.VectorSubcoreMesh(core_axis_name="c", subcore_axis_name="s")
    sc_fn = pl.kernel(
        _sc_body,
        mesh=mesh,
        compiler_params=pltpu.CompilerParams(needs_layout_passes=False),
        out_type=[
            jax.ShapeDtypeStruct((NC, NPAD, D), f32),
            jax.ShapeDtypeStruct((NC, NPAD), f32),
        ],
        scratch_types=[
            pltpu.VMEM((NPAD,), f32),
            pltpu.VMEM((NPAD,), f32),
            pltpu.VMEM((NRE, EB), i32),
            pltpu.VMEM((NRE, EB), i32),
            pltpu.VMEM((NRR, EB), f32),
            pltpu.VMEM((NRR, EB, D), f32),
            pltpu.VMEM_SHARED((NPAD, D), f32),
            pltpu.VMEM_SHARED((NPAD,), f32),
            pltpu.SemaphoreType.DMA((NRE,)),
            pltpu.SemaphoreType.DMA((NRE,)),
            pltpu.SemaphoreType.DMA((NRR,)),
            pltpu.SemaphoreType.DMA((NRR,)),
            pltpu.SemaphoreType.DMA((NRR,)),
        ],
    )
    hp_part, rs_part = sc_fn(ah, at, tail_val.astype(f32), src_p, dst_p)

    hp, rs = pl.pallas_call(
        _combine_body,
        grid=(NPAD // _CB,),
        in_specs=[
            pl.BlockSpec((NC, _CB, D), lambda i: (0, i, 0)),
            pl.BlockSpec((NC, _CB), lambda i: (0, i)),
        ],
        out_specs=[
            pl.BlockSpec((_CB, D), lambda i: (i, 0)),
            pl.BlockSpec((_CB, 1), lambda i: (i, 0)),
        ],
        out_shape=[
            jax.ShapeDtypeStruct((N_NODES, D), f32),
            jax.ShapeDtypeStruct((N_NODES, 1), f32),
        ],
    )(hp_part, rs_part)

    return (rs, hp)


# speedup vs baseline: 1.8740x; 1.8740x over previous
import jax
import jax.numpy as jnp
from jax import lax
from jax.experimental import pallas as pl
from jax.experimental.pallas import tpu as pltpu
from jax.experimental.pallas import tpu_sc as plsc

N_NODES = 10000
NPAD = 10240
N_EDGES = 320000
D = 128
ALPHA = 0.2
NC, NS, L = 2, 16, 16
NW = NC * NS
EB = 128
KS = 8
NI = 80
NSC = NI // KS
E_PAD = NI * NW * EB
NB_PAD = E_PAD // EB
TILE_ROWS = NPAD // NS
ROW_CHUNK = 128
N_CHUNKS = TILE_ROWS // ROW_CHUNK


def _scores_body(head_ref, tail_ref, attn_ref, ah_ref, at_ref):
    aw = attn_ref[...]
    ah_ref[...] = jnp.sum(head_ref[...] * aw[:, :D], axis=1, keepdims=True)
    at_ref[...] = jnp.sum(tail_ref[...] * aw[:, D:], axis=1, keepdims=True)


def _sc_body(ah_hbm, at_hbm, tv_hbm, src_hbm, dst_hbm,
             hp_out, rs_out,
             ah_tab, at_tab, src_sb, dst_sb, w_buf, rows,
             hp_acc, rs_acc, sem):
    c = lax.axis_index("c")
    s = lax.axis_index("s")
    wid = s * NC + c

    zero16 = jnp.zeros((L,), jnp.float32)

    def _zbody(r, carry):
        for j in range(D // L):
            rows[r, pl.ds(j * L, L)] = zero16
        return carry

    lax.fori_loop(0, ROW_CHUNK, _zbody, 0)

    tbase = s * TILE_ROWS
    for k in range(N_CHUNKS):
        pltpu.sync_copy(rows, hp_acc.at[pl.ds(tbase + k * ROW_CHUNK, ROW_CHUNK)])
        pltpu.sync_copy(rows.at[0], rs_acc.at[pl.ds(tbase + k * ROW_CHUNK, ROW_CHUNK)])

    pltpu.sync_copy(ah_hbm, ah_tab)
    pltpu.sync_copy(at_hbm, at_tab)

    plsc.subcore_barrier()

    sc0 = wid * NI

    def _sbody(j, carry):
        b0 = pl.multiple_of(sc0 + j * KS, KS)
        pltpu.sync_copy(src_hbm.at[pl.ds(b0, KS)], src_sb)
        pltpu.sync_copy(dst_hbm.at[pl.ds(b0, KS)], dst_sb)

        def _ebody(u, ecarry):
            pltpu.async_copy(tv_hbm.at[dst_sb.at[u]], rows, sem).wait()
            for jj in range(EB // L):
                si = src_sb[u, pl.ds(jj * L, L)]
                di = dst_sb[u, pl.ds(jj * L, L)]
                x = plsc.load_gather(ah_tab, [si]) + plsc.load_gather(at_tab, [di])
                x = jnp.clip(x, -10.0, 10.0)
                x = jnp.where(x >= 0.0, x, ALPHA * x)
                w_buf[pl.ds(jj * L, L)] = jnp.exp(x)

            def _mbody(g, mcarry):
                wv = w_buf[pl.ds(g * L, L)]
                for l in range(L):
                    wr = wv[l]
                    r = g * L + l
                    for kk in range(D // L):
                        rows[r, pl.ds(kk * L, L)] = rows[r, pl.ds(kk * L, L)] * wr
                return mcarry

            lax.fori_loop(0, EB // L, _mbody, 0)
            pltpu.sync_copy(rows, hp_acc.at[src_sb.at[u]], add=True)
            pltpu.sync_copy(w_buf, rs_acc.at[src_sb.at[u]], add=True)
            return ecarry

        lax.fori_loop(0, KS, _ebody, 0)
        return carry

    lax.fori_loop(0, NSC, _sbody, 0)

    plsc.subcore_barrier()

    pltpu.sync_copy(hp_acc.at[pl.ds(tbase, TILE_ROWS)],
                    hp_out.at[c, pl.ds(tbase, TILE_ROWS)])
    pltpu.sync_copy(rs_acc.at[pl.ds(tbase, TILE_ROWS)],
                    rs_out.at[c, pl.ds(tbase, TILE_ROWS)])


_CB = 1024


def _combine_body(hp_ref, rs_ref, hp_out_ref, rs_out_ref):
    hp_out_ref[...] = hp_ref[0] + hp_ref[1]
    rs_out_ref[...] = (rs_ref[0] + rs_ref[1])[:, None]


def kernel(head_rep, tail_rep, tail_val, edge_list, rel_list, attn):
    f32 = jnp.float32
    i32 = jnp.int32
    head_p = jnp.pad(head_rep.astype(f32), ((0, NPAD - N_NODES), (0, 0)))
    tail_p = jnp.pad(tail_rep.astype(f32), ((0, NPAD - N_NODES), (0, 0)))
    pad_n = E_PAD - N_EDGES
    src_p = jnp.concatenate([edge_list[0].astype(i32),
                             jnp.full((pad_n,), N_NODES, dtype=i32)])
    dst_p = jnp.concatenate([edge_list[1].astype(i32),
                             jnp.zeros((pad_n,), dtype=i32)])
    src2d = src_p.reshape(NB_PAD, EB)
    dst2d = dst_p.reshape(NB_PAD, EB)

    ah2, at2 = pl.pallas_call(
        _scores_body,
        grid=(NPAD // _CB,),
        in_specs=[
            pl.BlockSpec((_CB, D), lambda i: (i, 0)),
            pl.BlockSpec((_CB, D), lambda i: (i, 0)),
            pl.BlockSpec((1, 2 * D), lambda i: (0, 0)),
        ],
        out_specs=[
            pl.BlockSpec((_CB, 1), lambda i: (i, 0)),
            pl.BlockSpec((_CB, 1), lambda i: (i, 0)),
        ],
        out_shape=[
            jax.ShapeDtypeStruct((NPAD, 1), f32),
            jax.ShapeDtypeStruct((NPAD, 1), f32),
        ],
    )(head_p, tail_p, attn.astype(f32))
    ah = ah2.reshape(NPAD)
    at = at2.reshape(NPAD)

    mesh = plsc.VectorSubcoreMesh(core_axis_name="c", subcore_axis_name="s")
    sc_fn = pl.kernel(
        _sc_body,
        mesh=mesh,
        compiler_params=pltpu.CompilerParams(needs_layout_passes=False),
        out_type=[
            jax.ShapeDtypeStruct((NC, NPAD, D), f32),
            jax.ShapeDtypeStruct((NC, NPAD), f32),
        ],
        scratch_types=[
            pltpu.VMEM((NPAD,), f32),
            pltpu.VMEM((NPAD,), f32),
            pltpu.VMEM((KS, EB), i32),
            pltpu.VMEM((KS, EB), i32),
            pltpu.VMEM((EB,), f32),
            pltpu.VMEM((EB, D), f32),
            pltpu.VMEM_SHARED((NPAD, D), f32),
            pltpu.VMEM_SHARED((NPAD,), f32),
            pltpu.SemaphoreType.DMA,
        ],
    )
    hp_part, rs_part = sc_fn(ah, at, tail_val.astype(f32), src2d, dst2d)

    hp, rs = pl.pallas_call(
        _combine_body,
        grid=(NPAD // _CB,),
        in_specs=[
            pl.BlockSpec((NC, _CB, D), lambda i: (0, i, 0)),
            pl.BlockSpec((NC, _CB), lambda i: (0, i)),
        ],
        out_specs=[
            pl.BlockSpec((_CB, D), lambda i: (i, 0)),
            pl.BlockSpec((_CB, 1), lambda i: (i, 0)),
        ],
        out_shape=[
            jax.ShapeDtypeStruct((N_NODES, D), f32),
            jax.ShapeDtypeStruct((N_NODES, 1), f32),
        ],
    )(hp_part, rs_part)

    return (rs, hp)
